# Initial kernel scaffold; baseline (speedup 1.0000x reference)
import jax
import jax.numpy as jnp
from jax.experimental import pallas as pl


def kernel(x, pos_random, edge_index, gate_type, forward_index, num_nodes, W_rev, b_rev, W_v2c, b_v2c, Wu0, bu0, Wu1, bu1, Wu2, bu2, Wp0, bp0, Wp1, bp1, Wp2, bp2):
    raise NotImplementedError("write your pallas kernel here")


# speedup vs baseline: 11.8434x; 11.8434x over previous
import functools

import jax
import jax.numpy as jnp
from jax import lax
from jax.experimental import pallas as pl
from jax.experimental.pallas import tpu as pltpu
from jax.experimental.pallas import tpu_sc as plsc

NW = 32
LANES = 128


def _make_seg_sum(n_pad, width, nsteps, rps, nphases):
    mesh = plsc.VectorSubcoreMesh(core_axis_name="c", subcore_axis_name="s")

    @functools.partial(
        pl.kernel,
        mesh=mesh,
        compiler_params=pltpu.CompilerParams(use_tc_tiling_on_sc=False),
        out_type=jax.ShapeDtypeStruct((nphases, 2, n_pad, width), jnp.float32),
        scratch_types=[
            pltpu.VMEM((nsteps, LANES), jnp.int32),
            pltpu.VMEM((nsteps, LANES), jnp.int32),
            pltpu.VMEM((LANES, width), jnp.float32),
            pltpu.VMEM((LANES, width), jnp.float32),
            pltpu.VMEM_SHARED((n_pad, width), jnp.float32),
            pltpu.SemaphoreType.DMA,
            pltpu.SemaphoreType.DMA,
        ],
    )
    def seg(*refs):
        tables = refs[:nphases]
        (src_hbm, dst_hbm, zeros_hbm, out_hbm,
         src_v, dst_v, buf_a, buf_b, acc, sem_a, sem_b) = refs[nphases:]
        cid = lax.axis_index("c")
        sid = lax.axis_index("s")
        wid = sid * 2 + cid
        pltpu.sync_copy(src_hbm.at[wid], src_v)
        pltpu.sync_copy(dst_hbm.at[wid], dst_v)

        for h in range(nphases):
            table = tables[h]
            pltpu.sync_copy(zeros_hbm.at[pl.ds(sid * rps, rps)],
                            acc.at[pl.ds(sid * rps, rps)])
            plsc.subcore_barrier()

            pltpu.make_async_copy(table.at[src_v.at[0]], buf_a, sem_a).start()

            def body(jj, carry, table=table):
                j = jj * 2
                pltpu.make_async_copy(table.at[src_v.at[j + 1]], buf_b,
                                      sem_b).start()
                pltpu.make_async_copy(table.at[src_v.at[j]], buf_a,
                                      sem_a).wait()
                pltpu.sync_copy(buf_a, acc.at[dst_v.at[j]], add=True)

                @pl.when(jj + 1 < nsteps // 2)
                def _():
                    pltpu.make_async_copy(table.at[src_v.at[j + 2]], buf_a,
                                          sem_a).start()

                pltpu.make_async_copy(table.at[src_v.at[j + 1]], buf_b,
                                      sem_b).wait()
                pltpu.sync_copy(buf_b, acc.at[dst_v.at[j + 1]], add=True)
                return carry

            lax.fori_loop(0, nsteps // 2, body, 0)
            plsc.subcore_barrier()
            pltpu.sync_copy(acc.at[pl.ds(sid * rps, rps)],
                            out_hbm.at[h, cid, pl.ds(sid * rps, rps)])

    return seg


def _dot(a, b):
    return jnp.dot(a, b, preferred_element_type=jnp.float32)


def _pos_table_body(pos_ref, gate_ref, out_ref):
    g = gate_ref[...]
    sign = (g == 0).astype(jnp.float32) - (g == 1).astype(jnp.float32)
    lane = lax.broadcasted_iota(jnp.int32, out_ref.shape, 1)
    factor = jnp.where(lane < 16, sign, jnp.float32(1.0))
    out_ref[...] = pos_ref[...] * factor


def _node_state_body(q_ref, x_ref, gate_ref, wrev_ref, brev_ref,
                     wu0a_ref, wu0b_ref, bu0_ref, wu1_ref, bu1_ref,
                     wu2_ref, bu2_ref, lo_ref, hi_ref):
    q = q_ref[...]
    spos = q[0, 0, :, :16] + q[0, 1, :, :16]
    deg = q[0, 0, :, 16:17] + q[0, 1, :, 16:17]
    t = _dot(spos, wrev_ref[...]) + deg * brev_ref[...]
    h = jnp.maximum(_dot(t, wu0a_ref[...]) + _dot(x_ref[...], wu0b_ref[...])
                    + bu0_ref[...], 0.0)
    h = jnp.maximum(_dot(h, wu1_ref[...]) + bu1_ref[...], 0.0)
    u = _dot(h, wu2_ref[...]) + bu2_ref[...]
    m01 = (gate_ref[...] != 2).astype(jnp.float32)
    ns = u * m01
    lo_ref[...] = ns[:, :64]
    hi_ref[...] = ns[:, 64:]


def _final_body(p_ref, q_ref, x_ref, gate_ref, lo_ref, hi_ref,
                wv2c_ref, bv2c_ref,
                wu0a_ref, wu0b_ref, bu0_ref, wu1_ref, bu1_ref, wu2_ref, bu2_ref,
                wp0_ref, bp0_ref, wp1_ref, bp1_ref, wp2_ref, bp2_ref, out_ref):
    p = p_ref[...]
    q = q_ref[...]
    sns = jnp.concatenate([p[0, 0] + p[0, 1], p[1, 0] + p[1, 1]], axis=1)
    deg = q[0, 0, :, 16:17] + q[0, 1, :, 16:17]
    tc = _dot(sns, wv2c_ref[...]) + deg * bv2c_ref[...]
    h = jnp.maximum(_dot(tc, wu0a_ref[...]) + _dot(x_ref[...], wu0b_ref[...])
                    + bu0_ref[...], 0.0)
    h = jnp.maximum(_dot(h, wu1_ref[...]) + bu1_ref[...], 0.0)
    c = _dot(h, wu2_ref[...]) + bu2_ref[...]
    claf = (gate_ref[...] == 2).astype(jnp.float32)
    ns = jnp.concatenate([lo_ref[...], hi_ref[...]], axis=1)
    ns2 = c * claf + ns * (1.0 - claf)
    h = jnp.maximum(_dot(ns2, wp0_ref[...]) + bp0_ref[...], 0.0)
    h = jnp.maximum(_dot(h, wp1_ref[...]) + bp1_ref[...], 0.0)
    out_ref[...] = _dot(h, wp2_ref[...]) + bp2_ref[...]


def _full(shape):
    return pl.BlockSpec(shape, lambda i: (0,) * len(shape))


def _rows(shape, rowdim=0):
    def imap(i, rowdim=rowdim, rank=len(shape)):
        return tuple(i if d == rowdim else 0 for d in range(rank))
    return pl.BlockSpec(shape, imap)


def kernel(x, pos_random, edge_index, gate_type, forward_index, num_nodes,
           W_rev, b_rev, W_v2c, b_v2c,
           Wu0, bu0, Wu1, bu1, Wu2, bu2,
           Wp0, bp0, Wp1, bp1, Wp2, bp2):
    n = x.shape[0]
    e = edge_index.shape[1]
    dim_x = x.shape[1]
    dh = W_rev.shape[1]

    nsteps = -(-e // (NW * LANES))
    nsteps += nsteps % 2
    e_pad = NW * nsteps * LANES
    rps = -(-(n + 1) // 16)
    rps = -(-rps // 8) * 8
    n_pad = rps * 16

    src = edge_index[0].astype(jnp.int32)
    dst = edge_index[1].astype(jnp.int32)
    src = jnp.concatenate([src, jnp.zeros((e_pad - e,), jnp.int32)])
    dst = jnp.concatenate([dst, jnp.full((e_pad - e,), n, jnp.int32)])
    src_r = src.reshape(NW, nsteps, LANES)
    dst_r = dst.reshape(NW, nsteps, LANES)

    gate2 = gate_type.reshape(n, 1).astype(jnp.int32)
    pos24 = jnp.concatenate(
        [pos_random, jnp.ones((n, 1), jnp.float32),
         jnp.zeros((n, 7), jnp.float32)], axis=1)
    zeros24 = jnp.zeros((n_pad, 24), jnp.float32)
    zeros64 = jnp.zeros((n_pad, 64), jnp.float32)

    wu0a, wu0b = Wu0[:dh], Wu0[dh:]
    brev2 = b_rev.reshape(1, -1)
    bv2c2 = b_v2c.reshape(1, -1)
    bu02 = bu0.reshape(1, -1)
    bu12 = bu1.reshape(1, -1)
    bu22 = bu2.reshape(1, -1)
    bp02 = bp0.reshape(1, -1)
    bp12 = bp1.reshape(1, -1)
    bp22 = bp2.reshape(1, -1)

    rb = 1000
    grid = (n // rb,)

    pos_tab = pl.pallas_call(
        _pos_table_body,
        grid=grid,
        in_specs=[_rows((rb, 24)), _rows((rb, 1))],
        out_specs=_rows((rb, 24)),
        out_shape=jax.ShapeDtypeStruct((n, 24), jnp.float32),
    )(pos24, gate2)

    seg24 = _make_seg_sum(n_pad, 24, nsteps, rps, 1)
    q = seg24(pos_tab, src_r, dst_r, zeros24)

    ns_lo, ns_hi = pl.pallas_call(
        _node_state_body,
        grid=grid,
        in_specs=[
            _rows((1, 2, rb, 24), rowdim=2), _rows((rb, dim_x)), _rows((rb, 1)),
            _full((16, dh)), _full((1, dh)),
            _full((dh, Wu0.shape[1])), _full((dim_x, Wu0.shape[1])),
            _full((1, Wu0.shape[1])),
            _full(Wu1.shape), _full((1, Wu1.shape[1])),
            _full(Wu2.shape), _full((1, Wu2.shape[1])),
        ],
        out_specs=[_rows((rb, 64)), _rows((rb, 64))],
        out_shape=[jax.ShapeDtypeStruct((n, 64), jnp.float32),
                   jax.ShapeDtypeStruct((n, 64), jnp.float32)],
    )(q, x, gate2, W_rev, brev2, wu0a, wu0b, bu02, Wu1, bu12, Wu2, bu22)

    seg64 = _make_seg_sum(n_pad, 64, nsteps, rps, 2)
    p = seg64(ns_lo, ns_hi, src_r, dst_r, zeros64)

    out = pl.pallas_call(
        _final_body,
        grid=grid,
        in_specs=[
            _rows((2, 2, rb, 64), rowdim=2), _rows((1, 2, rb, 24), rowdim=2),
            _rows((rb, dim_x)), _rows((rb, 1)),
            _rows((rb, 64)), _rows((rb, 64)),
            _full(W_v2c.shape), _full((1, W_v2c.shape[1])),
            _full((dh, Wu0.shape[1])), _full((dim_x, Wu0.shape[1])),
            _full((1, Wu0.shape[1])),
            _full(Wu1.shape), _full((1, Wu1.shape[1])),
            _full(Wu2.shape), _full((1, Wu2.shape[1])),
            _full(Wp0.shape), _full((1, Wp0.shape[1])),
            _full(Wp1.shape), _full((1, Wp1.shape[1])),
            _full(Wp2.shape), _full((1, Wp2.shape[1])),
        ],
        out_specs=_rows((rb, Wp2.shape[1])),
        out_shape=jax.ShapeDtypeStruct((n, Wp2.shape[1]), jnp.float32),
    )(p, q, x, gate2, ns_lo, ns_hi, W_v2c, bv2c2, wu0a, wu0b, bu02,
      Wu1, bu12, Wu2, bu22, Wp0, bp02, Wp1, bp12, Wp2, bp22)

    return out


# speedup vs baseline: 11.9093x; 1.0056x over previous
import functools

import jax
import jax.numpy as jnp
from jax import lax
from jax.experimental import pallas as pl
from jax.experimental.pallas import tpu as pltpu
from jax.experimental.pallas import tpu_sc as plsc

NW = 32
LANES = 128


NBUF = 8


def _make_seg_sum(n_pad, width, nsteps, rps, nphases):
    mesh = plsc.VectorSubcoreMesh(core_axis_name="c", subcore_axis_name="s")
    ngroups = nsteps // NBUF

    @functools.partial(
        pl.kernel,
        mesh=mesh,
        compiler_params=pltpu.CompilerParams(use_tc_tiling_on_sc=False),
        out_type=jax.ShapeDtypeStruct((nphases, 2, n_pad, width), jnp.float32),
        scratch_types=[
            pltpu.VMEM((nsteps, LANES), jnp.int32),
            pltpu.VMEM((nsteps, LANES), jnp.int32),
        ] + [pltpu.VMEM((LANES, width), jnp.float32) for _ in range(NBUF)] + [
            pltpu.VMEM_SHARED((n_pad, width), jnp.float32),
        ] + [pltpu.SemaphoreType.DMA for _ in range(NBUF)],
    )
    def seg(*refs):
        tables = refs[:nphases]
        (src_hbm, dst_hbm, zeros_hbm, out_hbm, src_v, dst_v) = \
            refs[nphases:nphases + 6]
        bufs = refs[nphases + 6:nphases + 6 + NBUF]
        acc = refs[nphases + 6 + NBUF]
        sems = refs[nphases + 7 + NBUF:]
        cid = lax.axis_index("c")
        sid = lax.axis_index("s")
        wid = sid * 2 + cid
        pltpu.sync_copy(src_hbm.at[wid], src_v)
        pltpu.sync_copy(dst_hbm.at[wid], dst_v)

        for h in range(nphases):
            table = tables[h]

            def g_start(j, b, table=table):
                pltpu.async_copy(table.at[src_v.at[j]], bufs[b], sems[b])

            def g_wait(j, b, table=table):
                pltpu.make_async_copy(table.at[src_v.at[j]], bufs[b],
                                      sems[b]).wait()

            def s_start(j, b):
                pltpu.async_copy(bufs[b], acc.at[dst_v.at[j]], sems[b],
                                 add=True)

            def s_wait(j, b):
                pltpu.make_async_copy(bufs[b], acc.at[dst_v.at[j]],
                                      sems[b]).wait()

            pltpu.sync_copy(zeros_hbm.at[pl.ds(sid * rps, rps)],
                            acc.at[pl.ds(sid * rps, rps)])
            plsc.subcore_barrier()

            for idx in range(4):
                g_start(idx, idx)

            def body(k, carry):
                j0 = k * NBUF
                for idx in range(NBUF):
                    j = j0 + idx
                    bb = (idx + 4) % NBUF
                    if idx < 4:
                        @pl.when(k >= 1)
                        def _(j=j, bb=bb):
                            s_wait(j - 4, bb)
                        g_start(j + 4, bb)
                    else:
                        s_wait(j - 4, bb)

                        @pl.when(k < ngroups - 1)
                        def _(j=j, bb=bb):
                            g_start(j + 4, bb)
                    g_wait(j, idx)
                    s_start(j, idx)
                return carry

            lax.fori_loop(0, ngroups, body, 0)
            for idx in range(4):
                s_wait(nsteps - 4 + idx, idx + 4)
            plsc.subcore_barrier()
            pltpu.sync_copy(acc.at[pl.ds(sid * rps, rps)],
                            out_hbm.at[h, cid, pl.ds(sid * rps, rps)])

    return seg


def _dot(a, b):
    return jnp.dot(a, b, preferred_element_type=jnp.float32)


def _pos_table_body(pos_ref, gate_ref, out_ref):
    g = gate_ref[...]
    sign = (g == 0).astype(jnp.float32) - (g == 1).astype(jnp.float32)
    lane = lax.broadcasted_iota(jnp.int32, out_ref.shape, 1)
    factor = jnp.where(lane < 16, sign, jnp.float32(1.0))
    out_ref[...] = pos_ref[...] * factor


def _node_state_body(q_ref, x_ref, gate_ref, wrev_ref, brev_ref,
                     wu0a_ref, wu0b_ref, bu0_ref, wu1_ref, bu1_ref,
                     wu2_ref, bu2_ref, lo_ref, hi_ref):
    q = q_ref[...]
    spos = q[0, 0, :, :16] + q[0, 1, :, :16]
    deg = q[0, 0, :, 16:17] + q[0, 1, :, 16:17]
    t = _dot(spos, wrev_ref[...]) + deg * brev_ref[...]
    h = jnp.maximum(_dot(t, wu0a_ref[...]) + _dot(x_ref[...], wu0b_ref[...])
                    + bu0_ref[...], 0.0)
    h = jnp.maximum(_dot(h, wu1_ref[...]) + bu1_ref[...], 0.0)
    u = _dot(h, wu2_ref[...]) + bu2_ref[...]
    m01 = (gate_ref[...] != 2).astype(jnp.float32)
    ns = u * m01
    lo_ref[...] = ns[:, :64]
    hi_ref[...] = ns[:, 64:]


def _final_body(p_ref, q_ref, x_ref, gate_ref, lo_ref, hi_ref,
                wv2c_ref, bv2c_ref,
                wu0a_ref, wu0b_ref, bu0_ref, wu1_ref, bu1_ref, wu2_ref, bu2_ref,
                wp0_ref, bp0_ref, wp1_ref, bp1_ref, wp2_ref, bp2_ref, out_ref):
    p = p_ref[...]
    q = q_ref[...]
    sns = jnp.concatenate([p[0, 0] + p[0, 1], p[1, 0] + p[1, 1]], axis=1)
    deg = q[0, 0, :, 16:17] + q[0, 1, :, 16:17]
    tc = _dot(sns, wv2c_ref[...]) + deg * bv2c_ref[...]
    h = jnp.maximum(_dot(tc, wu0a_ref[...]) + _dot(x_ref[...], wu0b_ref[...])
                    + bu0_ref[...], 0.0)
    h = jnp.maximum(_dot(h, wu1_ref[...]) + bu1_ref[...], 0.0)
    c = _dot(h, wu2_ref[...]) + bu2_ref[...]
    claf = (gate_ref[...] == 2).astype(jnp.float32)
    ns = jnp.concatenate([lo_ref[...], hi_ref[...]], axis=1)
    ns2 = c * claf + ns * (1.0 - claf)
    h = jnp.maximum(_dot(ns2, wp0_ref[...]) + bp0_ref[...], 0.0)
    h = jnp.maximum(_dot(h, wp1_ref[...]) + bp1_ref[...], 0.0)
    out_ref[...] = _dot(h, wp2_ref[...]) + bp2_ref[...]


def _full(shape):
    return pl.BlockSpec(shape, lambda i: (0,) * len(shape))


def _rows(shape, rowdim=0):
    def imap(i, rowdim=rowdim, rank=len(shape)):
        return tuple(i if d == rowdim else 0 for d in range(rank))
    return pl.BlockSpec(shape, imap)


def kernel(x, pos_random, edge_index, gate_type, forward_index, num_nodes,
           W_rev, b_rev, W_v2c, b_v2c,
           Wu0, bu0, Wu1, bu1, Wu2, bu2,
           Wp0, bp0, Wp1, bp1, Wp2, bp2):
    n = x.shape[0]
    e = edge_index.shape[1]
    dim_x = x.shape[1]
    dh = W_rev.shape[1]

    nsteps = -(-e // (NW * LANES))
    nsteps = -(-nsteps // NBUF) * NBUF
    e_pad = NW * nsteps * LANES
    rps = -(-(n + 1) // 16)
    rps = -(-rps // 8) * 8
    n_pad = rps * 16

    src = edge_index[0].astype(jnp.int32)
    dst = edge_index[1].astype(jnp.int32)
    src = jnp.concatenate([src, jnp.zeros((e_pad - e,), jnp.int32)])
    dst = jnp.concatenate([dst, jnp.full((e_pad - e,), n, jnp.int32)])
    src_r = src.reshape(NW, nsteps, LANES)
    dst_r = dst.reshape(NW, nsteps, LANES)

    gate2 = gate_type.reshape(n, 1).astype(jnp.int32)
    pos24 = jnp.concatenate(
        [pos_random, jnp.ones((n, 1), jnp.float32),
         jnp.zeros((n, 7), jnp.float32)], axis=1)
    zeros24 = jnp.zeros((n_pad, 24), jnp.float32)
    zeros64 = jnp.zeros((n_pad, 64), jnp.float32)

    wu0a, wu0b = Wu0[:dh], Wu0[dh:]
    brev2 = b_rev.reshape(1, -1)
    bv2c2 = b_v2c.reshape(1, -1)
    bu02 = bu0.reshape(1, -1)
    bu12 = bu1.reshape(1, -1)
    bu22 = bu2.reshape(1, -1)
    bp02 = bp0.reshape(1, -1)
    bp12 = bp1.reshape(1, -1)
    bp22 = bp2.reshape(1, -1)

    rb = 1000
    grid = (n // rb,)

    pos_tab = pl.pallas_call(
        _pos_table_body,
        grid=grid,
        in_specs=[_rows((rb, 24)), _rows((rb, 1))],
        out_specs=_rows((rb, 24)),
        out_shape=jax.ShapeDtypeStruct((n, 24), jnp.float32),
    )(pos24, gate2)

    seg24 = _make_seg_sum(n_pad, 24, nsteps, rps, 1)
    q = seg24(pos_tab, src_r, dst_r, zeros24)

    ns_lo, ns_hi = pl.pallas_call(
        _node_state_body,
        grid=grid,
        in_specs=[
            _rows((1, 2, rb, 24), rowdim=2), _rows((rb, dim_x)), _rows((rb, 1)),
            _full((16, dh)), _full((1, dh)),
            _full((dh, Wu0.shape[1])), _full((dim_x, Wu0.shape[1])),
            _full((1, Wu0.shape[1])),
            _full(Wu1.shape), _full((1, Wu1.shape[1])),
            _full(Wu2.shape), _full((1, Wu2.shape[1])),
        ],
        out_specs=[_rows((rb, 64)), _rows((rb, 64))],
        out_shape=[jax.ShapeDtypeStruct((n, 64), jnp.float32),
                   jax.ShapeDtypeStruct((n, 64), jnp.float32)],
    )(q, x, gate2, W_rev, brev2, wu0a, wu0b, bu02, Wu1, bu12, Wu2, bu22)

    seg64 = _make_seg_sum(n_pad, 64, nsteps, rps, 2)
    p = seg64(ns_lo, ns_hi, src_r, dst_r, zeros64)

    out = pl.pallas_call(
        _final_body,
        grid=grid,
        in_specs=[
            _rows((2, 2, rb, 64), rowdim=2), _rows((1, 2, rb, 24), rowdim=2),
            _rows((rb, dim_x)), _rows((rb, 1)),
            _rows((rb, 64)), _rows((rb, 64)),
            _full(W_v2c.shape), _full((1, W_v2c.shape[1])),
            _full((dh, Wu0.shape[1])), _full((dim_x, Wu0.shape[1])),
            _full((1, Wu0.shape[1])),
            _full(Wu1.shape), _full((1, Wu1.shape[1])),
            _full(Wu2.shape), _full((1, Wu2.shape[1])),
            _full(Wp0.shape), _full((1, Wp0.shape[1])),
            _full(Wp1.shape), _full((1, Wp1.shape[1])),
            _full(Wp2.shape), _full((1, Wp2.shape[1])),
        ],
        out_specs=_rows((rb, Wp2.shape[1])),
        out_shape=jax.ShapeDtypeStruct((n, Wp2.shape[1]), jnp.float32),
    )(p, q, x, gate2, ns_lo, ns_hi, W_v2c, bv2c2, wu0a, wu0b, bu02,
      Wu1, bu12, Wu2, bu22, Wp0, bp02, Wp1, bp12, Wp2, bp22)

    return out
